# Initial kernel scaffold; baseline (speedup 1.0000x reference)
import jax
import jax.numpy as jnp
from jax.experimental import pallas as pl


def kernel(xyz, times, feat, W0, b0, g0, be0, m0, v0, W1, b1, g1, be1, m1, v1):
    raise NotImplementedError("write your pallas kernel here")


# speedup vs baseline: 1.0001x; 1.0001x over previous
import jax
import jax.numpy as jnp
from jax.experimental import pallas as pl

NPOINT_ = 1024
K_ = 32


def _fps(pts, npoint):
    pts = jax.lax.stop_gradient(pts)
    Bt, N, _ = pts.shape

    def body(i, state):
        dist, farthest, idxs = state
        idxs = idxs.at[:, i].set(farthest)
        centroid = jnp.take_along_axis(pts, farthest[:, None, None], axis=1)
        d = jnp.sum((pts - centroid) ** 2, axis=-1)
        dist = jnp.minimum(dist, d)
        farthest = jnp.argmax(dist, axis=-1).astype(jnp.int32)
        return (dist, farthest, idxs)

    init = (jnp.full((Bt, N), 1e10, jnp.float32), jnp.zeros((Bt,), jnp.int32),
            jnp.zeros((Bt, npoint), jnp.int32))
    _, _, idxs = jax.lax.fori_loop(0, npoint, body, init)
    return idxs


def _gather_points(p, idx):
    return jnp.take_along_axis(p, idx[:, None, :], axis=2)


def _group_gather(p, idx):
    return jax.vmap(lambda f, i: f[:, i])(p, idx)


def _knn_idx(support, queries, k):
    s = jax.lax.stop_gradient(support)
    q = jax.lax.stop_gradient(queries)
    d2 = (jnp.sum(q * q, axis=1)[:, :, None] + jnp.sum(s * s, axis=1)[:, None, :]
          - 2.0 * jnp.einsum('bcm,bcn->bmn', q, s))
    _, idx = jax.lax.top_k(-d2, k)
    return idx


def _conv_bn_lrelu(x, W, b, g, be, m, v):
    y = jnp.einsum('oi,bimk->bomk', W, x) + b[None, :, None, None]
    y = (y - m[None, :, None, None]) / jnp.sqrt(v[None, :, None, None] + 1e-3) * g[None, :, None, None] + be[None, :, None, None]
    return jax.nn.leaky_relu(y, 0.01)


def kernel(xyz, times, feat, W0, b0, g0, be0, m0, v0, W1, b1, g1, be1, m1, v1):
    B, T, _, N = xyz.shape
    C = feat.shape[1]
    xyz_flat = xyz.reshape(B * T, 3, N)
    idx = _fps(jnp.transpose(xyz_flat, (0, 2, 1)), NPOINT_)
    xyz1 = _gather_points(xyz_flat, idx)
    points = jnp.transpose(xyz1.reshape(B, T, 3, NPOINT_), (0, 2, 1, 3)).reshape(B, 3, T * NPOINT_)
    feat_frames = []
    for j in range(T):
        support = xyz[:, j]
        nidx = _knn_idx(support, points, K_)
        xyz_g = _group_gather(support, nidx)
        f_g = _group_gather(feat[:, :, j, :], nidx)
        xyz_diff = xyz_g - points[:, :, :, None]
        fg = jnp.concatenate([xyz_diff, f_g], axis=1)
        h = _conv_bn_lrelu(fg, W0, b0, g0, be0, m0, v0)
        h = _conv_bn_lrelu(h, W1, b1, g1, be1, m1, v1)
        feat_frames.append(jnp.max(h, axis=-1))
    out = jnp.max(jnp.stack(feat_frames, axis=1), axis=1)
    return out


# speedup vs baseline: 1.3169x; 1.3168x over previous
import functools

import jax
import jax.numpy as jnp
from jax.experimental import pallas as pl
from jax.experimental.pallas import tpu as pltpu

NPOINT_ = 1024
K_ = 32
N_ = 4096


def _fps_body(xs_ref, ys_ref, zs_ref, o_ref):
    x = xs_ref[0]
    y = ys_ref[0]
    z = zs_ref[0]
    R = x.shape[0]
    lidx = jax.lax.broadcasted_iota(jnp.int32, (R, N_), 1)
    piota = jax.lax.broadcasted_iota(jnp.int32, (R, NPOINT_), 1)

    def step(i, carry):
        dist, far, ax, ay, az = carry
        sel = lidx == far
        cx = jnp.sum(jnp.where(sel, x, 0.0), axis=1, keepdims=True)
        cy = jnp.sum(jnp.where(sel, y, 0.0), axis=1, keepdims=True)
        cz = jnp.sum(jnp.where(sel, z, 0.0), axis=1, keepdims=True)
        hit = piota == i
        ax = jnp.where(hit, cx, ax)
        ay = jnp.where(hit, cy, ay)
        az = jnp.where(hit, cz, az)
        dx = x - cx
        dy = y - cy
        dz = z - cz
        d = (dx * dx + dy * dy) + dz * dz
        dist = jnp.minimum(dist, d)
        m = jnp.max(dist, axis=1, keepdims=True)
        far = jnp.min(jnp.where(dist >= m, lidx, jnp.int32(N_)), axis=1,
                      keepdims=True)
        return dist, far, ax, ay, az

    dist0 = jnp.full((R, N_), 1e10, jnp.float32)
    far0 = jnp.zeros((R, 1), jnp.int32)
    acc0 = jnp.zeros((R, NPOINT_), jnp.float32)
    _, _, ax, ay, az = jax.lax.fori_loop(0, NPOINT_, step,
                                         (dist0, far0, acc0, acc0, acc0))
    o_ref[0, 0] = ax
    o_ref[1, 0] = ay
    o_ref[2, 0] = az


def _fps_points(xyz_flat, interpret=False):
    xs = xyz_flat[:, 0, :].reshape(2, 3, N_)
    ys = xyz_flat[:, 1, :].reshape(2, 3, N_)
    zs = xyz_flat[:, 2, :].reshape(2, 3, N_)
    out = pl.pallas_call(
        _fps_body,
        grid=(2,),
        in_specs=[pl.BlockSpec((1, 3, N_), lambda i: (i, 0, 0))] * 3,
        out_specs=pl.BlockSpec((3, 1, 3, NPOINT_), lambda i: (0, i, 0, 0)),
        out_shape=jax.ShapeDtypeStruct((3, 2, 3, NPOINT_), jnp.float32),
        compiler_params=pltpu.CompilerParams(
            dimension_semantics=("parallel",)),
        interpret=interpret,
    )(xs, ys, zs)
    return out


def _knn_idx(support, queries, k):
    d2 = (jnp.sum(queries * queries, axis=1)[:, :, None]
          + jnp.sum(support * support, axis=1)[:, None, :]
          - 2.0 * jnp.einsum('bcm,bcn->bmn', queries, support))
    _, idx = jax.lax.top_k(-d2, k)
    return idx


def _group_gather(p, idx):
    return jax.vmap(lambda f, i: f[:, i])(p, idx)


def _conv_bn_lrelu(x, W, b, g, be, m, v):
    y = jnp.einsum('oi,bimk->bomk', W, x) + b[None, :, None, None]
    y = (y - m[None, :, None, None]) / jnp.sqrt(v[None, :, None, None] + 1e-3) * g[None, :, None, None] + be[None, :, None, None]
    return jax.nn.leaky_relu(y, 0.01)


def kernel(xyz, times, feat, W0, b0, g0, be0, m0, v0, W1, b1, g1, be1, m1, v1):
    B, T, _, N = xyz.shape
    xyz_flat = xyz.reshape(B * T, 3, N)
    psel = _fps_points(xyz_flat)
    points = jnp.transpose(psel, (1, 0, 2, 3)).reshape(B, 3, T * NPOINT_)
    feat_frames = []
    for j in range(T):
        support = xyz[:, j]
        nidx = _knn_idx(support, points, K_)
        xyz_g = _group_gather(support, nidx)
        f_g = _group_gather(feat[:, :, j, :], nidx)
        xyz_diff = xyz_g - points[:, :, :, None]
        fg = jnp.concatenate([xyz_diff, f_g], axis=1)
        h = _conv_bn_lrelu(fg, W0, b0, g0, be0, m0, v0)
        h = _conv_bn_lrelu(h, W1, b1, g1, be1, m1, v1)
        feat_frames.append(jnp.max(h, axis=-1))
    out = jnp.max(jnp.stack(feat_frames, axis=1), axis=1)
    return out


# speedup vs baseline: 10.7259x; 8.1449x over previous
import functools

import jax
import jax.numpy as jnp
from jax.experimental import pallas as pl
from jax.experimental.pallas import tpu as pltpu
from jax.experimental.pallas import tpu_sc as plsc

NPOINT_ = 1024
K_ = 32
N_ = 4096
QB_ = 512
CPAD_ = 128


def _fps_body(xs_ref, ys_ref, zs_ref, o_ref):
    x = xs_ref[0]
    y = ys_ref[0]
    z = zs_ref[0]
    R = x.shape[0]
    lidx = jax.lax.broadcasted_iota(jnp.int32, (R, N_), 1)
    piota = jax.lax.broadcasted_iota(jnp.int32, (R, NPOINT_), 1)

    def step(i, carry):
        dist, far, ax, ay, az = carry
        sel = lidx == far
        cx = jnp.sum(jnp.where(sel, x, 0.0), axis=1, keepdims=True)
        cy = jnp.sum(jnp.where(sel, y, 0.0), axis=1, keepdims=True)
        cz = jnp.sum(jnp.where(sel, z, 0.0), axis=1, keepdims=True)
        hit = piota == i
        ax = jnp.where(hit, cx, ax)
        ay = jnp.where(hit, cy, ay)
        az = jnp.where(hit, cz, az)
        dx = x - cx
        dy = y - cy
        dz = z - cz
        d = (dx * dx + dy * dy) + dz * dz
        dist = jnp.minimum(dist, d)
        m = jnp.max(dist, axis=1, keepdims=True)
        far = jnp.min(jnp.where(dist >= m, lidx, jnp.int32(N_)), axis=1,
                      keepdims=True)
        return dist, far, ax, ay, az

    dist0 = jnp.full((R, N_), 1e10, jnp.float32)
    far0 = jnp.zeros((R, 1), jnp.int32)
    acc0 = jnp.zeros((R, NPOINT_), jnp.float32)
    _, _, ax, ay, az = jax.lax.fori_loop(0, NPOINT_, step,
                                         (dist0, far0, acc0, acc0, acc0))
    o_ref[0, 0] = ax
    o_ref[1, 0] = ay
    o_ref[2, 0] = az


def _fps_points(xyz_flat, interpret=False):
    xs = xyz_flat[:, 0, :].reshape(2, 3, N_)
    ys = xyz_flat[:, 1, :].reshape(2, 3, N_)
    zs = xyz_flat[:, 2, :].reshape(2, 3, N_)
    return pl.pallas_call(
        _fps_body,
        grid=(2,),
        in_specs=[pl.BlockSpec((1, 3, N_), lambda i: (i, 0, 0))] * 3,
        out_specs=pl.BlockSpec((3, 1, 3, NPOINT_), lambda i: (0, i, 0, 0)),
        out_shape=jax.ShapeDtypeStruct((3, 2, 3, NPOINT_), jnp.float32),
        compiler_params=pltpu.CompilerParams(
            dimension_semantics=("parallel",)),
        interpret=interpret,
    )(xs, ys, zs)


def _topk_body(supp_ref, q_ref, o_ref):
    s = supp_ref[0]
    qT = q_ref[0]
    bt = pl.program_id(0)
    s2 = jnp.sum(s * s, axis=0, keepdims=True)
    qq = jnp.sum(qT * qT, axis=1, keepdims=True)
    dot = jnp.dot(qT, s, preferred_element_type=jnp.float32)
    d2 = (qq + s2) - 2.0 * dot
    lidx = jax.lax.broadcasted_iota(jnp.int32, (QB_, N_), 1)
    cols = []
    for _ in range(K_):
        m = jnp.min(d2, axis=1, keepdims=True)
        am = jnp.min(jnp.where(d2 <= m, lidx, jnp.int32(N_)), axis=1,
                     keepdims=True)
        cols.append(am)
        d2 = jnp.where(lidx == am, jnp.float32(jnp.inf), d2)
    o_ref[0] = jnp.concatenate(cols, axis=1) + bt * N_


def _knn_topk(xyz_flat, points_t, interpret=False):
    TQ = points_t.shape[1]
    return pl.pallas_call(
        _topk_body,
        grid=(6, TQ // QB_),
        in_specs=[
            pl.BlockSpec((1, 3, N_), lambda bt, q: (bt, 0, 0)),
            pl.BlockSpec((1, QB_, 3), lambda bt, q: (bt // 3, q, 0)),
        ],
        out_specs=pl.BlockSpec((1, QB_, K_), lambda bt, q: (bt, q, 0)),
        out_shape=jax.ShapeDtypeStruct((6, TQ, K_), jnp.int32),
        compiler_params=pltpu.CompilerParams(
            dimension_semantics=("parallel", "parallel")),
        interpret=interpret,
    )(xyz_flat, points_t)


def _sc_gather(tab, nidx_flat):
    NI = nidx_flat.shape[0]
    W = 128
    idx2 = nidx_flat.reshape(1, NI)

    @pl.kernel(
        out_type=jax.ShapeDtypeStruct((NI, CPAD_), jnp.float32),
        mesh=plsc.VectorSubcoreMesh(core_axis_name="core",
                                    subcore_axis_name="subcore"),
    )
    def gat(tab_hbm, idx_hbm, o_hbm):
        def body(i_vmem, o_vmem):
            pltpu.sync_copy(tab_hbm.at[i_vmem.at[0]], o_vmem)

        pltpu.emit_pipeline(
            body,
            grid=(NI // W,),
            in_specs=[pl.BlockSpec((1, W), lambda i: (0, i))],
            out_specs=[pl.BlockSpec((W, CPAD_), lambda i: (i, 0))],
            core_axis_name=("core", "subcore"),
            dimension_semantics=(pltpu.PARALLEL,),
        )(idx_hbm, o_hbm)

    return gat(tab, idx2)


def _mlp_body(g_ref, q_ref, w0_ref, c0_ref, w1_ref, c1_ref, o_ref):
    X = g_ref[0]
    qT = q_ref[0]
    w0 = w0_ref[...]
    w1 = w1_ref[...]
    XW = jnp.dot(X, w0, preferred_element_type=jnp.float32)
    cq = c0_ref[...] - jnp.dot(qT, w0[0:3, :],
                               preferred_element_type=jnp.float32)
    h1 = XW.reshape(QB_, K_, 64) + cq[:, None, :]
    h1 = jnp.where(h1 >= 0, h1, 0.01 * h1)
    h2 = jnp.dot(h1.reshape(QB_ * K_, 64), w1,
                 preferred_element_type=jnp.float32) + c1_ref[...]
    h2 = jnp.where(h2 >= 0, h2, 0.01 * h2)
    r = jnp.max(h2.reshape(QB_, K_, 128), axis=1)
    t = pl.program_id(2)

    @pl.when(t == 0)
    def _():
        o_ref[0] = r

    @pl.when(t != 0)
    def _():
        o_ref[0] = jnp.maximum(o_ref[0], r)


def _mlp_max(gath, points_t, w0p, c0, w1p, c1, B, T, interpret=False):
    TQ = points_t.shape[1]
    NQ = TQ // QB_
    return pl.pallas_call(
        _mlp_body,
        grid=(B, NQ, T),
        in_specs=[
            pl.BlockSpec((1, QB_ * K_, CPAD_),
                         lambda b, q, t: (b * T + t, q, 0)),
            pl.BlockSpec((1, QB_, 3), lambda b, q, t: (b, q, 0)),
            pl.BlockSpec((CPAD_, 64), lambda b, q, t: (0, 0)),
            pl.BlockSpec((1, 64), lambda b, q, t: (0, 0)),
            pl.BlockSpec((64, 128), lambda b, q, t: (0, 0)),
            pl.BlockSpec((1, 128), lambda b, q, t: (0, 0)),
        ],
        out_specs=pl.BlockSpec((1, QB_, 128), lambda b, q, t: (b, q, 0)),
        out_shape=jax.ShapeDtypeStruct((B, TQ, 128), jnp.float32),
        compiler_params=pltpu.CompilerParams(
            dimension_semantics=("parallel", "parallel", "arbitrary")),
        interpret=interpret,
    )(gath, points_t, w0p, c0, w1p, c1)


def kernel(xyz, times, feat, W0, b0, g0, be0, m0, v0, W1, b1, g1, be1, m1, v1):
    B, T, _, N = xyz.shape
    C = feat.shape[1]
    TQ = T * NPOINT_

    s0 = g0 / jnp.sqrt(v0 + 1e-3)
    c0 = (b0 * s0 + be0 - m0 * s0).reshape(1, 64)
    W0p = W0 * s0[:, None]
    w0p = jnp.zeros((CPAD_, 64), jnp.float32).at[: 3 + C, :].set(W0p.T)
    s1 = g1 / jnp.sqrt(v1 + 1e-3)
    c1 = (b1 * s1 + be1 - m1 * s1).reshape(1, 128)
    w1p = (W1 * s1[:, None]).T

    xyz_flat = xyz.reshape(B * T, 3, N)

    psel = _fps_points(xyz_flat)
    points_t = jnp.transpose(psel, (1, 2, 3, 0)).reshape(B, TQ, 3)

    nidx = _knn_topk(xyz_flat, points_t)

    featT = jnp.transpose(feat, (0, 2, 1, 3))
    ptab = jnp.concatenate([xyz, featT], axis=2)
    ptab = jnp.transpose(ptab, (0, 1, 3, 2)).reshape(B * T * N, 3 + C)
    ptab = jnp.pad(ptab, ((0, 0), (0, CPAD_ - 3 - C)))
    gath = _sc_gather(ptab, nidx.reshape(-1))
    gath = gath.reshape(B * T, TQ * K_, CPAD_)

    out = _mlp_max(gath, points_t, w0p, c0, w1p, c1, B, T)
    return jnp.transpose(out, (0, 2, 1))
